# baseline (speedup 1.0000x reference)
import functools

import jax
import jax.numpy as jnp
from jax import lax
from jax.experimental import pallas as pl
from jax.experimental.pallas import tpu as pltpu
from jax.experimental.pallas import tpu_sc as plsc

_NC = 2
_NS = 16
_NW = _NC * _NS


def _sc_gather(pe_table, ids):
    v, d = pe_table.shape
    b = ids.shape[0]
    b_per_w = b // _NW

    mesh = plsc.VectorSubcoreMesh(core_axis_name="c", subcore_axis_name="s")

    @functools.partial(
        pl.kernel,
        mesh=mesh,
        out_type=jax.ShapeDtypeStruct((b, d), jnp.float32),
        scratch_types=[
            pltpu.VMEM((b_per_w,), jnp.int32),
            pltpu.VMEM((b_per_w, d), jnp.float32),
            pltpu.SemaphoreType.DMA,
        ],
    )
    def gather_kernel(table_hbm, idx_hbm, out_hbm, idx_v, rows_v, sem):
        wid = lax.axis_index("s") * _NC + lax.axis_index("c")
        base = wid * b_per_w
        pltpu.sync_copy(idx_hbm.at[pl.ds(base, b_per_w)], idx_v)
        pltpu.async_copy(table_hbm.at[idx_v], rows_v, sem).wait()
        pltpu.sync_copy(rows_v, out_hbm.at[pl.ds(base, b_per_w)])

    return gather_kernel(pe_table, ids)


def _tc_add_body(pe_ref, x_ref, o_ref, pet_ref):
    @pl.when(pl.program_id(0) == 0)
    def _():
        pet_ref[...] = jnp.swapaxes(pe_ref[...], 0, 1)

    o_ref[...] = x_ref[...] + pet_ref[...][None]


def _tc_add(x2, pe_g):
    b, ch, n_pos = x2.shape
    return pl.pallas_call(
        _tc_add_body,
        grid=(b,),
        in_specs=[
            pl.BlockSpec((n_pos, ch), lambda i: (0, 0)),
            pl.BlockSpec((1, ch, n_pos), lambda i: (i, 0, 0)),
        ],
        out_specs=pl.BlockSpec((1, ch, n_pos), lambda i: (i, 0, 0)),
        out_shape=jax.ShapeDtypeStruct((b, ch, n_pos), jnp.float32),
        scratch_shapes=[pltpu.VMEM((ch, n_pos), jnp.float32)],
    )(pe_g, x2)


def kernel(x, pe_table, position_ids):
    b, ch, h, w = x.shape
    n_pos = h * w
    ids = position_ids.reshape(n_pos).astype(jnp.int32)
    pe_g = _sc_gather(pe_table, ids)
    x2 = x.reshape(b, ch, n_pos)
    out2 = _tc_add(x2, pe_g)
    return out2.reshape(b, ch, h, w)


# speedup vs baseline: 3.2529x; 3.2529x over previous
import functools

import jax
import jax.numpy as jnp
from jax import lax
from jax.experimental import pallas as pl
from jax.experimental.pallas import tpu as pltpu
from jax.experimental.pallas import tpu_sc as plsc

_NC = 2
_NS = 16
_NW = _NC * _NS


def _sc_gather(pe_table, ids):
    v, d = pe_table.shape
    b = ids.shape[0]
    b_per_w = b // _NW

    mesh = plsc.VectorSubcoreMesh(core_axis_name="c", subcore_axis_name="s")

    @functools.partial(
        pl.kernel,
        mesh=mesh,
        out_type=jax.ShapeDtypeStruct((b, d), jnp.float32),
        scratch_types=[
            pltpu.VMEM((b_per_w,), jnp.int32),
            pltpu.VMEM((b_per_w, d), jnp.float32),
            pltpu.SemaphoreType.DMA,
        ],
    )
    def gather_kernel(table_hbm, idx_hbm, out_hbm, idx_v, rows_v, sem):
        wid = lax.axis_index("s") * _NC + lax.axis_index("c")
        base = wid * b_per_w
        pltpu.sync_copy(idx_hbm.at[pl.ds(base, b_per_w)], idx_v)
        pltpu.async_copy(table_hbm.at[idx_v], rows_v, sem).wait()
        pltpu.sync_copy(rows_v, out_hbm.at[pl.ds(base, b_per_w)])

    return gather_kernel(pe_table, ids)


def _tc_add_body(pe_ref, x_ref, o_ref):
    o_ref[...] = x_ref[...] + pe_ref[...][None]


def _tc_add(x2t, pe_g):
    b, n_pos, ch = x2t.shape
    return pl.pallas_call(
        _tc_add_body,
        grid=(b,),
        in_specs=[
            pl.BlockSpec((n_pos, ch), lambda i: (0, 0)),
            pl.BlockSpec((1, n_pos, ch), lambda i: (i, 0, 0)),
        ],
        out_specs=pl.BlockSpec((1, n_pos, ch), lambda i: (i, 0, 0)),
        out_shape=jax.ShapeDtypeStruct((b, n_pos, ch), jnp.float32),
    )(pe_g, x2t)


def kernel(x, pe_table, position_ids):
    b, ch, h, w = x.shape
    n_pos = h * w
    ids = position_ids.reshape(n_pos).astype(jnp.int32)
    pe_g = _sc_gather(pe_table, ids)
    x2t = jnp.transpose(x, (0, 2, 3, 1)).reshape(b, n_pos, ch)
    out2t = _tc_add(x2t, pe_g)
    return jnp.transpose(out2t.reshape(b, h, w, ch), (0, 3, 1, 2))


# speedup vs baseline: 3.3368x; 1.0258x over previous
import functools

import jax
import jax.numpy as jnp
from jax import lax
from jax.experimental import pallas as pl
from jax.experimental.pallas import tpu as pltpu
from jax.experimental.pallas import tpu_sc as plsc

_NC = 2
_NS = 16
_NW = _NC * _NS


def _sc_gather(pe_table, ids):
    v, d = pe_table.shape
    b = ids.shape[0]
    b_per_w = b // _NW

    mesh = plsc.VectorSubcoreMesh(core_axis_name="c", subcore_axis_name="s")

    @functools.partial(
        pl.kernel,
        mesh=mesh,
        out_type=jax.ShapeDtypeStruct((b, d), jnp.float32),
        scratch_types=[
            pltpu.VMEM((b_per_w,), jnp.int32),
            pltpu.VMEM((b_per_w, d), jnp.float32),
            pltpu.SemaphoreType.DMA,
        ],
    )
    def gather_kernel(table_hbm, idx_hbm, out_hbm, idx_v, rows_v, sem):
        wid = lax.axis_index("s") * _NC + lax.axis_index("c")
        base = wid * b_per_w
        pltpu.sync_copy(idx_hbm.at[pl.ds(base, b_per_w)], idx_v)
        pltpu.async_copy(table_hbm.at[idx_v], rows_v, sem).wait()
        pltpu.sync_copy(rows_v, out_hbm.at[pl.ds(base, b_per_w)])

    return gather_kernel(pe_table, ids)


def _tc_add_body(pe_ref, x_ref, o_ref):
    o_ref[...] = x_ref[...] + pe_ref[...][None]


_BB = 2


def _tc_add(x2t, pe_g):
    b, n_pos, ch = x2t.shape
    return pl.pallas_call(
        _tc_add_body,
        grid=(b // _BB,),
        in_specs=[
            pl.BlockSpec((n_pos, ch), lambda i: (0, 0)),
            pl.BlockSpec((_BB, n_pos, ch), lambda i: (i, 0, 0)),
        ],
        out_specs=pl.BlockSpec((_BB, n_pos, ch), lambda i: (i, 0, 0)),
        out_shape=jax.ShapeDtypeStruct((b, n_pos, ch), jnp.float32),
    )(pe_g, x2t)


def kernel(x, pe_table, position_ids):
    b, ch, h, w = x.shape
    n_pos = h * w
    ids = position_ids.reshape(n_pos).astype(jnp.int32)
    pe_g = _sc_gather(pe_table, ids)
    x2t = jnp.transpose(x, (0, 2, 3, 1)).reshape(b, n_pos, ch)
    out2t = _tc_add(x2t, pe_g)
    return jnp.transpose(out2t.reshape(b, h, w, ch), (0, 3, 1, 2))


# speedup vs baseline: 3.3582x; 1.0064x over previous
import functools

import jax
import jax.numpy as jnp
from jax import lax
from jax.experimental import pallas as pl
from jax.experimental.pallas import tpu as pltpu
from jax.experimental.pallas import tpu_sc as plsc

_NC = 2
_NS = 16
_NW = _NC * _NS


def _sc_gather(pe_table, ids):
    v, d = pe_table.shape
    b = ids.shape[0]
    b_per_w = b // _NW

    mesh = plsc.VectorSubcoreMesh(core_axis_name="c", subcore_axis_name="s")

    @functools.partial(
        pl.kernel,
        mesh=mesh,
        out_type=jax.ShapeDtypeStruct((b, d), jnp.float32),
        scratch_types=[
            pltpu.VMEM((b_per_w,), jnp.int32),
            pltpu.VMEM((b_per_w, d), jnp.float32),
            pltpu.SemaphoreType.DMA,
        ],
    )
    def gather_kernel(table_hbm, idx_hbm, out_hbm, idx_v, rows_v, sem):
        wid = lax.axis_index("s") * _NC + lax.axis_index("c")
        base = wid * b_per_w
        pltpu.sync_copy(idx_hbm.at[pl.ds(base, b_per_w)], idx_v)
        pltpu.async_copy(table_hbm.at[idx_v], rows_v, sem).wait()
        pltpu.sync_copy(rows_v, out_hbm.at[pl.ds(base, b_per_w)])

    return gather_kernel(pe_table, ids)


def _tc_add_body(pe_ref, x_ref, o_ref):
    o_ref[...] = x_ref[...] + pe_ref[...][None]


_BB = 4


def _tc_add(x2t, pe_g):
    b, n_pos, ch = x2t.shape
    return pl.pallas_call(
        _tc_add_body,
        grid=(b // _BB,),
        in_specs=[
            pl.BlockSpec((n_pos, ch), lambda i: (0, 0)),
            pl.BlockSpec((_BB, n_pos, ch), lambda i: (i, 0, 0)),
        ],
        out_specs=pl.BlockSpec((_BB, n_pos, ch), lambda i: (i, 0, 0)),
        out_shape=jax.ShapeDtypeStruct((b, n_pos, ch), jnp.float32),
    )(pe_g, x2t)


def kernel(x, pe_table, position_ids):
    b, ch, h, w = x.shape
    n_pos = h * w
    ids = position_ids.reshape(n_pos).astype(jnp.int32)
    pe_g = _sc_gather(pe_table, ids)
    x2t = jnp.transpose(x, (0, 2, 3, 1)).reshape(b, n_pos, ch)
    out2t = _tc_add(x2t, pe_g)
    return jnp.transpose(out2t.reshape(b, h, w, ch), (0, 3, 1, 2))
